# Initial kernel scaffold; baseline (speedup 1.0000x reference)
import jax
import jax.numpy as jnp
from jax.experimental import pallas as pl


def kernel(x, edge_index, W, b):
    raise NotImplementedError("write your pallas kernel here")


# speedup vs baseline: 1693.5852x; 1693.5852x over previous
import functools

import jax
import jax.numpy as jnp
from jax import lax
from jax.experimental import pallas as pl
from jax.experimental.pallas import tpu as pltpu
from jax.experimental.pallas import tpu_sc as plsc

B, C, N, K, COUT = 2, 128, 10000, 16, 128
NB = 1000
NW = 32
NPW = (B * N) // NW
G = 5
EPG = G * K
STEPS = NPW // G
LANES = 16


def _tc_tables_body(x_ref, w_ref, b_ref, y1_ref, y2_ref):
    xb = x_ref[0]
    w2 = w_ref[:, C:]
    a = w_ref[:, :C] - w2
    dn = (((1,), (1,)), ((), ()))
    y1_ref[...] = lax.dot_general(xb, a, dn, preferred_element_type=jnp.float32)
    y2_ref[...] = (lax.dot_general(xb, w2, dn, preferred_element_type=jnp.float32)
                   + b_ref[...])


_tc_tables = pl.pallas_call(
    _tc_tables_body,
    grid=(B, N // NB),
    in_specs=[
        pl.BlockSpec((1, NB, C), lambda b, j: (b, j, 0)),
        pl.BlockSpec((COUT, 2 * C), lambda b, j: (0, 0)),
        pl.BlockSpec((1, COUT), lambda b, j: (0, 0)),
    ],
    out_specs=[
        pl.BlockSpec((NB, COUT), lambda b, j: (b * (N // NB) + j, 0)),
        pl.BlockSpec((NB, COUT), lambda b, j: (b * (N // NB) + j, 0)),
    ],
    out_shape=[
        jax.ShapeDtypeStruct((B * N, COUT), jnp.float32),
        jax.ShapeDtypeStruct((B * N, COUT), jnp.float32),
    ],
)


@functools.partial(
    pl.kernel,
    mesh=plsc.VectorSubcoreMesh(core_axis_name="c", subcore_axis_name="s"),
    out_type=jax.ShapeDtypeStruct((B * N * COUT,), jnp.float32),
    scratch_types=[
        pltpu.VMEM((EPG,), jnp.int32),
        pltpu.VMEM((EPG,), jnp.int32),
        pltpu.VMEM((EPG, COUT), jnp.float32),
        pltpu.VMEM((EPG, COUT), jnp.float32),
        pltpu.VMEM((G * COUT,), jnp.float32),
        pltpu.SemaphoreType.DMA,
        pltpu.SemaphoreType.DMA,
    ],
)
def _sc_edge_max(y1_hbm, y2_hbm, i1_hbm, i0_hbm, out_hbm,
                 i1v, i0v, r1v, r2v, outv, sem1, sem2):
    wid = lax.axis_index("s") * 2 + lax.axis_index("c")
    node_base = wid * NPW
    edge_base = node_base * K

    def step(s, carry):
        eoff = edge_base + s * EPG
        pltpu.sync_copy(i1_hbm.at[pl.ds(eoff, EPG)], i1v)
        pltpu.sync_copy(i0_hbm.at[pl.ds(eoff, EPG)], i0v)
        cp1 = pltpu.async_copy(y1_hbm.at[i1v], r1v, sem1)
        cp2 = pltpu.async_copy(y2_hbm.at[i0v], r2v, sem2)
        cp1.wait()
        cp2.wait()
        for g in range(G):
            for cb in range(COUT // LANES):
                co = cb * LANES
                acc = r1v[g * K, pl.ds(co, LANES)] + r2v[g * K, pl.ds(co, LANES)]
                for k in range(1, K):
                    acc = jnp.maximum(
                        acc,
                        r1v[g * K + k, pl.ds(co, LANES)]
                        + r2v[g * K + k, pl.ds(co, LANES)])
                outv[pl.ds(g * COUT + co, LANES)] = jnp.maximum(acc, 0.0)
        pltpu.sync_copy(outv,
                        out_hbm.at[pl.ds((node_base + s * G) * COUT, G * COUT)])
        return carry

    lax.fori_loop(0, STEPS, step, 0)


def kernel(x, edge_index, W, b):
    xT = jnp.transpose(x[..., 0], (0, 2, 1))
    y1, y2 = _tc_tables(xT, W, b.reshape(1, COUT))
    offs = (jnp.arange(B, dtype=jnp.int32) * N).reshape(B, 1, 1)
    i1f = (edge_index[1] + offs).reshape(-1)
    i0f = (edge_index[0] + offs).reshape(-1)
    out = _sc_edge_max(y1, y2, i1f, i0f)
    return out.reshape(B, N, COUT).transpose(0, 2, 1)[..., None]


# speedup vs baseline: 2559.0353x; 1.5110x over previous
import functools

import jax
import jax.numpy as jnp
from jax import lax
from jax.experimental import pallas as pl
from jax.experimental.pallas import tpu as pltpu
from jax.experimental.pallas import tpu_sc as plsc

B, C, N, K, COUT = 2, 128, 10000, 16, 128
NB = 1000
NW = 32
NPW = (B * N) // NW
G = 5
EPG = G * K
STEPS = NPW // G
LANES = 16


def _tc_tables_body(x_ref, w_ref, b_ref, y1_ref, y2_ref):
    xb = x_ref[0]
    w2 = w_ref[:, C:]
    a = w_ref[:, :C] - w2
    dn = (((1,), (1,)), ((), ()))
    y1_ref[...] = lax.dot_general(xb, a, dn, preferred_element_type=jnp.float32)
    y2_ref[...] = (lax.dot_general(xb, w2, dn, preferred_element_type=jnp.float32)
                   + b_ref[...])


_tc_tables = pl.pallas_call(
    _tc_tables_body,
    grid=(B, N // NB),
    in_specs=[
        pl.BlockSpec((1, NB, C), lambda b, j: (b, j, 0)),
        pl.BlockSpec((COUT, 2 * C), lambda b, j: (0, 0)),
        pl.BlockSpec((1, COUT), lambda b, j: (0, 0)),
    ],
    out_specs=[
        pl.BlockSpec((NB, COUT), lambda b, j: (b * (N // NB) + j, 0)),
        pl.BlockSpec((NB, COUT), lambda b, j: (b * (N // NB) + j, 0)),
    ],
    out_shape=[
        jax.ShapeDtypeStruct((B * N, COUT), jnp.float32),
        jax.ShapeDtypeStruct((B * N, COUT), jnp.float32),
    ],
)


@functools.partial(
    pl.kernel,
    mesh=plsc.VectorSubcoreMesh(core_axis_name="c", subcore_axis_name="s"),
    out_type=jax.ShapeDtypeStruct((B * N * COUT,), jnp.float32),
    scratch_types=[
        pltpu.VMEM((NPW * K,), jnp.int32),
        pltpu.VMEM((NPW * K,), jnp.int32),
        pltpu.VMEM((EPG, COUT), jnp.float32),
        pltpu.VMEM((EPG, COUT), jnp.float32),
        pltpu.VMEM((EPG, COUT), jnp.float32),
        pltpu.VMEM((EPG, COUT), jnp.float32),
        pltpu.VMEM((G * COUT,), jnp.float32),
        pltpu.VMEM((G * COUT,), jnp.float32),
        pltpu.SemaphoreType.DMA,
        pltpu.SemaphoreType.DMA,
        pltpu.SemaphoreType.DMA,
        pltpu.SemaphoreType.DMA,
        pltpu.SemaphoreType.DMA,
        pltpu.SemaphoreType.DMA,
    ],
)
def _sc_edge_max(y1_hbm, y2_hbm, i1_hbm, i0_hbm, out_hbm,
                 i1v, i0v, r1a, r1b, r2a, r2b, outa, outb,
                 sg1a, sg1b, sg2a, sg2b, soa, sob):
    wid = lax.axis_index("s") * 2 + lax.axis_index("c")
    node_base = wid * NPW
    edge_base = node_base * K

    slots = ((r1a, r2a, outa, sg1a, sg2a, soa),
             (r1b, r2b, outb, sg1b, sg2b, sob))

    def issue(s, slot):
        r1, r2, _, s1, s2, _ = slots[slot]
        pltpu.async_copy(y1_hbm.at[i1v.at[pl.ds(s * EPG, EPG)]], r1, s1)
        pltpu.async_copy(y2_hbm.at[i0v.at[pl.ds(s * EPG, EPG)]], r2, s2)

    def wait_gathers(slot):
        r1, r2, _, s1, s2, _ = slots[slot]
        pltpu.make_async_copy(y1_hbm.at[i1v.at[pl.ds(0, EPG)]], r1, s1).wait()
        pltpu.make_async_copy(y2_hbm.at[i0v.at[pl.ds(0, EPG)]], r2, s2).wait()

    def drain_store(slot):
        _, _, outv, _, _, so = slots[slot]
        pltpu.make_async_copy(outv, out_hbm.at[pl.ds(0, G * COUT)], so).wait()

    def compute_store(s, slot):
        r1, r2, outv, _, _, so = slots[slot]
        for g in range(G):
            for cb in range(COUT // LANES):
                co = cb * LANES
                acc = r1[g * K, pl.ds(co, LANES)] + r2[g * K, pl.ds(co, LANES)]
                for k in range(1, K):
                    acc = jnp.maximum(
                        acc,
                        r1[g * K + k, pl.ds(co, LANES)]
                        + r2[g * K + k, pl.ds(co, LANES)])
                outv[pl.ds(g * COUT + co, LANES)] = jnp.maximum(acc, 0.0)
        pltpu.async_copy(
            outv, out_hbm.at[pl.ds((node_base + s * G) * COUT, G * COUT)], so)

    pltpu.sync_copy(i1_hbm.at[pl.ds(edge_base, NPW * K)], i1v)
    pltpu.sync_copy(i0_hbm.at[pl.ds(edge_base, NPW * K)], i0v)

    issue(0, 0)

    def body(t, carry):
        s0 = 2 * t
        issue(s0 + 1, 1)
        wait_gathers(0)

        @pl.when(t > 0)
        def _():
            drain_store(0)
        compute_store(s0, 0)
        issue(s0 + 2, 0)
        wait_gathers(1)

        @pl.when(t > 0)
        def _():
            drain_store(1)
        compute_store(s0 + 1, 1)
        return carry

    lax.fori_loop(0, (STEPS - 1) // 2, body, 0)

    wait_gathers(0)
    drain_store(0)
    compute_store(STEPS - 1, 0)
    drain_store(0)
    drain_store(1)


def kernel(x, edge_index, W, b):
    xT = jnp.transpose(x[..., 0], (0, 2, 1))
    y1, y2 = _tc_tables(xT, W, b.reshape(1, COUT))
    offs = (jnp.arange(B, dtype=jnp.int32) * N).reshape(B, 1, 1)
    i1f = (edge_index[1] + offs).reshape(-1)
    i0f = (edge_index[0] + offs).reshape(-1)
    out = _sc_edge_max(y1, y2, i1f, i0f)
    return out.reshape(B, N, COUT).transpose(0, 2, 1)[..., None]
